# Initial kernel scaffold; baseline (speedup 1.0000x reference)
import jax
import jax.numpy as jnp
from jax.experimental import pallas as pl


def kernel(target_feat, edge_index_mp0, edge_index_mp1, W0, b0, a0, W1, b1, a1, att_fc_W, att_fc_b, att_v, attcl_fc_W, attcl_fc_b, attcl_v):
    raise NotImplementedError("write your pallas kernel here")


# speedup vs baseline: 12.5715x; 12.5715x over previous
import functools

import jax
import jax.numpy as jnp
from jax import lax
from jax.experimental import pallas as pl
from jax.experimental.pallas import tpu as pltpu, tpu_sc as plsc

N = 10000
E = 320000
H = 128
GROUP = 2000
NP = 10240
NS = 16
PER_TILE = E // NS
KB = 2
CW = 80
CHUNK = KB * CW
N_ITERS = PER_TILE // CHUNK
N_CHUNKS = E // CHUNK
ROWS_T = NP // NS

_mesh = plsc.VectorSubcoreMesh(core_axis_name="c", subcore_axis_name="s",
                               num_cores=2, num_subcores=NS)


def _sc_degrees(edges_r, zeros1d):
    @functools.partial(
        pl.kernel,
        out_type=jax.ShapeDtypeStruct((2, 2, NP), jnp.float32),
        mesh=_mesh,
        scratch_types=[
            pltpu.VMEM((KB, CW), jnp.int32),
            pltpu.VMEM((KB, CW), jnp.int32),
            pltpu.VMEM((CW,), jnp.float32),
            pltpu.VMEM_SHARED((NP,), jnp.float32),
            pltpu.VMEM_SHARED((NP,), jnp.float32),
        ],
    )
    def deg_kernel(edges, zeros_h, out, sidx, didx, ones, deg_s, deg_d):
        c = lax.axis_index("c")
        s = lax.axis_index("s")
        for j in range(CW // 16):
            ones[pl.ds(j * 16, 16)] = jnp.ones((16,), jnp.float32)
        seg = NP // NS
        pltpu.sync_copy(zeros_h, deg_s.at[pl.ds(seg * s, seg)])
        pltpu.sync_copy(zeros_h, deg_d.at[pl.ds(seg * s, seg)])
        plsc.subcore_barrier()

        def graph_loop(g):
            def body(i, carry):
                chunk = s * N_ITERS + i
                pltpu.sync_copy(edges.at[g, 0, chunk], sidx)
                pltpu.sync_copy(edges.at[g, 1, chunk], didx)
                for j in range(KB):
                    pltpu.sync_copy(ones, deg_s.at[sidx.at[j]], add=True)
                    pltpu.sync_copy(ones, deg_d.at[didx.at[j]], add=True)
                return carry

            lax.fori_loop(0, N_ITERS, body, 0)

        for g in (0, 1):
            @pl.when(c == g)
            def _():
                graph_loop(g)

        plsc.subcore_barrier()
        for g in (0, 1):
            @pl.when(c == g)
            def _():
                pltpu.sync_copy(deg_s.at[pl.ds(seg * s, seg)],
                                out.at[g, 0, pl.ds(seg * s, seg)])
                pltpu.sync_copy(deg_d.at[pl.ds(seg * s, seg)],
                                out.at[g, 1, pl.ds(seg * s, seg)])

    return deg_kernel(edges_r, zeros1d)


def _sc_agg(edges_r, hs0, hs1, zrows):
    @functools.partial(
        pl.kernel,
        out_type=[jax.ShapeDtypeStruct((NP, H), jnp.float32)] * 2,
        mesh=_mesh,
        scratch_types=[
            pltpu.VMEM((KB, CW), jnp.int32),
            pltpu.VMEM((KB, CW), jnp.int32),
            pltpu.VMEM((KB, CW, H), jnp.float32),
            pltpu.VMEM_SHARED((NP, H), jnp.float32),
            pltpu.SemaphoreType.DMA,
        ],
    )
    def agg_kernel(edges, h0, h1, zr, out0, out1, sidx, didx, rows, accum, sem):
        c = lax.axis_index("c")
        s = lax.axis_index("s")
        pltpu.sync_copy(zr, accum.at[pl.ds(ROWS_T * s, ROWS_T)])
        plsc.subcore_barrier()

        def graph_loop(g, h_ref):
            def body(i, carry):
                chunk = s * N_ITERS + i
                pltpu.sync_copy(edges.at[g, 0, chunk], sidx)
                pltpu.sync_copy(edges.at[g, 1, chunk], didx)
                descs = [pltpu.async_copy(h_ref.at[sidx.at[j]], rows.at[j], sem)
                         for j in range(KB)]
                for d in descs:
                    d.wait()
                for j in range(KB):
                    pltpu.sync_copy(rows.at[j], accum.at[didx.at[j]], add=True)
                return carry

            lax.fori_loop(0, N_ITERS, body, 0)

        for g, h_ref in ((0, h0), (1, h1)):
            @pl.when(c == g)
            def _():
                graph_loop(g, h_ref)

        plsc.subcore_barrier()
        for g, o_ref in ((0, out0), (1, out1)):
            @pl.when(c == g)
            def _():
                pltpu.sync_copy(accum.at[pl.ds(ROWS_T * s, ROWS_T)],
                                o_ref.at[pl.ds(ROWS_T * s, ROWS_T)])

    return agg_kernel(edges_r, hs0, hs1, zrows)


def _tc_hs(x, W0, W1, ds0, ds1):
    def body(x_ref, w0_ref, w1_ref, d0_ref, d1_ref, o0_ref, o1_ref):
        xv = x_ref[...]
        for d_ref, w_ref, o_ref in ((d0_ref, w0_ref, o0_ref),
                                    (d1_ref, w1_ref, o1_ref)):
            d = d_ref[...]
            ns = jnp.where(d > 0, lax.rsqrt(jnp.maximum(d, 1.0)), 0.0)
            o_ref[...] = jnp.dot(xv * ns, w_ref[...],
                                 preferred_element_type=jnp.float32)

    return pl.pallas_call(
        body,
        out_shape=[jax.ShapeDtypeStruct((N, H), jnp.float32)] * 2,
    )(x, W0, W1, ds0, ds1)


def _tc_tail(agg0, agg1, dd0, dd1, b0, a0, b1, a1,
             fcWT, fcb, v, clWT, clb, clv):
    def body(a0_ref, a1_ref, d0_ref, d1_ref, b0_ref, s0_ref, b1_ref, s1_ref,
             fw_ref, fb_ref, v_ref, cw_ref, cb_ref, cv_ref, out_ref):
        def conv_out(ag_ref, d_ref, b_ref, slope_ref):
            d = d_ref[...]
            nd = jnp.where(d > 0, lax.rsqrt(jnp.maximum(d, 1.0)), 0.0)
            y = ag_ref[...] * nd + b_ref[...]
            return jnp.where(y >= 0, y, slope_ref[...] * y)

        e0 = conv_out(a0_ref, d0_ref, b0_ref, s0_ref)
        e1 = conv_out(a1_ref, d1_ref, b1_ref, s1_ref)
        fw = fw_ref[...]
        fb = fb_ref[...]
        vv = v_ref[...]
        inv_n = jnp.float32(1.0 / N)
        l0 = jnp.sum(jnp.tanh(jnp.dot(e0, fw, preferred_element_type=jnp.float32)
                              + fb) * vv) * inv_n
        l1 = jnp.sum(jnp.tanh(jnp.dot(e1, fw, preferred_element_type=jnp.float32)
                              + fb) * vv) * inv_n
        m = jnp.maximum(l0, l1)
        w0 = jnp.exp(l0 - m)
        w1 = jnp.exp(l1 - m)
        inv_sum = 1.0 / (w0 + w1)
        z = (w0 * inv_sum) * e0 + (w1 * inv_sum) * e1

        tz = jnp.tanh(jnp.dot(z, cw_ref[...], preferred_element_type=jnp.float32)
                      + cb_ref[...])
        cvv = cv_ref[...]
        inv_g = jnp.float32(1.0 / GROUP)
        gl = [jnp.sum(tz[i * GROUP:(i + 1) * GROUP] * cvv) * inv_g
              for i in range(N // GROUP)]
        gm = gl[0]
        for t in gl[1:]:
            gm = jnp.maximum(gm, t)
        gw = [jnp.exp(t - gm) for t in gl]
        gsum = gw[0]
        for t in gw[1:]:
            gsum = gsum + t
        inv_gsum = 1.0 / gsum
        acc = (gw[0] * inv_gsum) * z[0:GROUP]
        for i in range(1, N // GROUP):
            acc = acc + (gw[i] * inv_gsum) * z[i * GROUP:(i + 1) * GROUP]
        out_ref[...] = acc

    return pl.pallas_call(
        body,
        out_shape=jax.ShapeDtypeStruct((GROUP, H), jnp.float32),
    )(agg0, agg1, dd0, dd1, b0, a0, b1, a1, fcWT, fcb, v, clWT, clb, clv)


def kernel(target_feat, edge_index_mp0, edge_index_mp1, W0, b0, a0, W1, b1, a1,
           att_fc_W, att_fc_b, att_v, attcl_fc_W, attcl_fc_b, attcl_v):
    edges = jnp.stack([edge_index_mp0.astype(jnp.int32),
                       edge_index_mp1.astype(jnp.int32)])
    edges_r = edges.reshape(2, 2, N_CHUNKS, KB, CW)

    zeros1d = jnp.zeros((NP // NS,), jnp.float32)
    degs = _sc_degrees(edges_r, zeros1d)
    ds0 = degs[0, 0, :N].reshape(N, 1)
    dd0 = degs[0, 1, :N].reshape(N, 1)
    ds1 = degs[1, 0, :N].reshape(N, 1)
    dd1 = degs[1, 1, :N].reshape(N, 1)

    hs0, hs1 = _tc_hs(target_feat, W0, W1, ds0, ds1)

    zrows = jnp.zeros((ROWS_T, H), jnp.float32)
    agg0, agg1 = _sc_agg(edges_r, hs0, hs1, zrows)
    agg0 = agg0[:N]
    agg1 = agg1[:N]

    out = _tc_tail(
        agg0, agg1, dd0, dd1,
        b0.reshape(1, H), a0.reshape(1, 1), b1.reshape(1, H), a1.reshape(1, 1),
        att_fc_W.T, att_fc_b.reshape(1, H), att_v.reshape(1, H),
        attcl_fc_W.T, attcl_fc_b.reshape(1, H), attcl_v.reshape(1, H))
    return out


# speedup vs baseline: 26.9079x; 2.1404x over previous
import functools

import jax
import jax.numpy as jnp
from jax import lax
from jax.experimental import pallas as pl
from jax.experimental.pallas import tpu as pltpu, tpu_sc as plsc

N = 10000
E = 320000
H = 128
GROUP = 2000
NP = 10240
NS = 16
PER_TILE = E // NS
KB = 2
CW = 80
CHUNK = KB * CW
N_ITERS = PER_TILE // CHUNK
N_CHUNKS = E // CHUNK
SLAB = PER_TILE // CW
ROWS_T = NP // NS

_mesh = plsc.VectorSubcoreMesh(core_axis_name="c", subcore_axis_name="s",
                               num_cores=2, num_subcores=NS)


def _sc_degrees(edges_d, zeros1d):
    @functools.partial(
        pl.kernel,
        out_type=jax.ShapeDtypeStruct((2, 2, NP), jnp.float32),
        mesh=_mesh,
        scratch_types=[
            pltpu.VMEM((SLAB, CW), jnp.int32),
            pltpu.VMEM((SLAB, CW), jnp.int32),
            pltpu.VMEM((CW,), jnp.float32),
            pltpu.VMEM_SHARED((NP,), jnp.float32),
            pltpu.VMEM_SHARED((NP,), jnp.float32),
            pltpu.SemaphoreType.DMA,
            pltpu.SemaphoreType.DMA,
        ],
    )
    def deg_kernel(edges, zeros_h, out, sslab, dslab, ones, deg_s, deg_d,
                   sem_i, sem_s):
        c = lax.axis_index("c")
        s = lax.axis_index("s")
        for j in range(CW // 16):
            ones[pl.ds(j * 16, 16)] = jnp.ones((16,), jnp.float32)
        seg = NP // NS
        pltpu.sync_copy(zeros_h, deg_s.at[pl.ds(seg * s, seg)])
        pltpu.sync_copy(zeros_h, deg_d.at[pl.ds(seg * s, seg)])
        plsc.subcore_barrier()

        def graph_run(g):
            pltpu.async_copy(edges.at[g, 0, s], sslab, sem_i)
            pltpu.async_copy(edges.at[g, 1, s], dslab, sem_i)
            pltpu.make_async_copy(edges.at[g, 0, s], sslab, sem_i).wait()
            pltpu.make_async_copy(edges.at[g, 0, s], dslab, sem_i).wait()

            def fire(k):
                pltpu.async_copy(ones, deg_s.at[sslab.at[k]], sem_s, add=True)
                pltpu.async_copy(ones, deg_d.at[dslab.at[k]], sem_s, add=True)

            def drain(k):
                pltpu.make_async_copy(ones, deg_s.at[sslab.at[k]], sem_s).wait()
                pltpu.make_async_copy(ones, deg_d.at[dslab.at[k]], sem_s).wait()

            depth = 4
            for k in range(depth):
                fire(k)

            def body(k, carry):
                fire(k)
                drain(k - depth)
                return carry

            lax.fori_loop(depth, SLAB, body, 0)
            for k in range(depth):
                drain(SLAB - depth + k)

        for g in (0, 1):
            @pl.when(c == g)
            def _():
                graph_run(g)

        plsc.subcore_barrier()
        for g in (0, 1):
            @pl.when(c == g)
            def _():
                pltpu.sync_copy(deg_s.at[pl.ds(seg * s, seg)],
                                out.at[g, 0, pl.ds(seg * s, seg)])
                pltpu.sync_copy(deg_d.at[pl.ds(seg * s, seg)],
                                out.at[g, 1, pl.ds(seg * s, seg)])

    return deg_kernel(edges_d, zeros1d)


def _sc_agg(edges_r, hs0, hs1, zrows):
    @functools.partial(
        pl.kernel,
        out_type=[jax.ShapeDtypeStruct((NP, H), jnp.float32)] * 2,
        mesh=_mesh,
        scratch_types=[
            pltpu.VMEM((KB, CW), jnp.int32),
            pltpu.VMEM((KB, CW), jnp.int32),
            pltpu.VMEM((KB, CW), jnp.int32),
            pltpu.VMEM((KB, CW), jnp.int32),
            pltpu.VMEM((KB, CW, H), jnp.float32),
            pltpu.VMEM((KB, CW, H), jnp.float32),
            pltpu.VMEM_SHARED((NP, H), jnp.float32),
            pltpu.SemaphoreType.DMA,
            pltpu.SemaphoreType.DMA,
            pltpu.SemaphoreType.DMA,
        ],
    )
    def agg_kernel(edges, h0, h1, zr, out0, out1,
                   sidx0, didx0, sidx1, didx1, rows0, rows1, accum,
                   sem_idx, sem_gat, sem_sct):
        c = lax.axis_index("c")
        s = lax.axis_index("s")
        pltpu.sync_copy(zr, accum.at[pl.ds(ROWS_T * s, ROWS_T)])
        plsc.subcore_barrier()

        def run_graph(g, h_ref):
            def fire_idx(i, sb, db):
                chunk = s * N_ITERS + i
                pltpu.async_copy(edges.at[g, 0, chunk], sb, sem_idx)
                pltpu.async_copy(edges.at[g, 1, chunk], db, sem_idx)

            def wait_idx(sb, db):
                pltpu.make_async_copy(edges.at[g, 0, 0], sb, sem_idx).wait()
                pltpu.make_async_copy(edges.at[g, 0, 0], db, sem_idx).wait()

            def fire_gat(sb, rb):
                for j in range(KB):
                    pltpu.async_copy(h_ref.at[sb.at[j]], rb.at[j], sem_gat)

            def wait_gat(sb, rb):
                for j in range(KB):
                    pltpu.make_async_copy(h_ref.at[sb.at[j]], rb.at[j],
                                          sem_gat).wait()

            def fire_sct(db, rb):
                for j in range(KB):
                    pltpu.async_copy(rb.at[j], accum.at[db.at[j]], sem_sct,
                                     add=True)

            def wait_sct(db, rb):
                for j in range(KB):
                    pltpu.make_async_copy(rb.at[j], accum.at[db.at[j]],
                                          sem_sct).wait()

            B0 = (sidx0, didx0, rows0)
            B1 = (sidx1, didx1, rows1)

            fire_idx(0, sidx0, didx0)
            wait_idx(sidx0, didx0)
            fire_gat(sidx0, rows0)
            fire_idx(1, sidx1, didx1)

            def iter_body(i, pb, qb):
                (sp, dp, rp), (sq, dq, rq) = pb, qb
                wait_idx(sp, dp)
                wait_gat(sq, rq)
                fire_gat(sp, rp)
                fire_sct(dq, rq)
                wait_sct(dq, rq)
                fire_idx(jnp.minimum(i + 1, N_ITERS - 1), sq, dq)

            def pair(k, carry):
                i = 2 * k + 1
                iter_body(i, B1, B0)
                iter_body(i + 1, B0, B1)
                return carry

            lax.fori_loop(0, (N_ITERS - 1) // 2, pair, 0)

            wait_idx(sidx1, didx1)
            wait_gat(sidx0, rows0)
            fire_sct(didx0, rows0)
            wait_sct(didx0, rows0)

        for g, h_ref in ((0, h0), (1, h1)):
            @pl.when(c == g)
            def _():
                run_graph(g, h_ref)

        plsc.subcore_barrier()
        for g, o_ref in ((0, out0), (1, out1)):
            @pl.when(c == g)
            def _():
                pltpu.sync_copy(accum.at[pl.ds(ROWS_T * s, ROWS_T)],
                                o_ref.at[pl.ds(ROWS_T * s, ROWS_T)])

    return agg_kernel(edges_r, hs0, hs1, zrows)


def _tc_hs(x, W0, W1, ds0, ds1):
    def body(x_ref, w0_ref, w1_ref, d0_ref, d1_ref, o0_ref, o1_ref):
        xv = x_ref[...]
        for d_ref, w_ref, o_ref in ((d0_ref, w0_ref, o0_ref),
                                    (d1_ref, w1_ref, o1_ref)):
            d = d_ref[...]
            ns = jnp.where(d > 0, lax.rsqrt(jnp.maximum(d, 1.0)), 0.0)
            o_ref[...] = jnp.dot(xv * ns, w_ref[...],
                                 preferred_element_type=jnp.float32)

    return pl.pallas_call(
        body,
        out_shape=[jax.ShapeDtypeStruct((N, H), jnp.float32)] * 2,
    )(x, W0, W1, ds0, ds1)


def _tc_tail(agg0, agg1, dd0, dd1, b0, a0, b1, a1,
             fcWT, fcb, v, clWT, clb, clv):
    def body(a0_ref, a1_ref, d0_ref, d1_ref, b0_ref, s0_ref, b1_ref, s1_ref,
             fw_ref, fb_ref, v_ref, cw_ref, cb_ref, cv_ref, out_ref):
        def conv_out(ag_ref, d_ref, b_ref, slope_ref):
            d = d_ref[...]
            nd = jnp.where(d > 0, lax.rsqrt(jnp.maximum(d, 1.0)), 0.0)
            y = ag_ref[0:N] * nd + b_ref[...]
            return jnp.where(y >= 0, y, slope_ref[...] * y)

        e0 = conv_out(a0_ref, d0_ref, b0_ref, s0_ref)
        e1 = conv_out(a1_ref, d1_ref, b1_ref, s1_ref)
        fw = fw_ref[...]
        fb = fb_ref[...]
        vv = v_ref[...]
        inv_n = jnp.float32(1.0 / N)
        l0 = jnp.sum(jnp.tanh(jnp.dot(e0, fw, preferred_element_type=jnp.float32)
                              + fb) * vv) * inv_n
        l1 = jnp.sum(jnp.tanh(jnp.dot(e1, fw, preferred_element_type=jnp.float32)
                              + fb) * vv) * inv_n
        m = jnp.maximum(l0, l1)
        w0 = jnp.exp(l0 - m)
        w1 = jnp.exp(l1 - m)
        inv_sum = 1.0 / (w0 + w1)
        z = (w0 * inv_sum) * e0 + (w1 * inv_sum) * e1

        tz = jnp.tanh(jnp.dot(z, cw_ref[...], preferred_element_type=jnp.float32)
                      + cb_ref[...])
        cvv = cv_ref[...]
        inv_g = jnp.float32(1.0 / GROUP)
        gl = [jnp.sum(tz[i * GROUP:(i + 1) * GROUP] * cvv) * inv_g
              for i in range(N // GROUP)]
        gm = gl[0]
        for t in gl[1:]:
            gm = jnp.maximum(gm, t)
        gw = [jnp.exp(t - gm) for t in gl]
        gsum = gw[0]
        for t in gw[1:]:
            gsum = gsum + t
        inv_gsum = 1.0 / gsum
        acc = (gw[0] * inv_gsum) * z[0:GROUP]
        for i in range(1, N // GROUP):
            acc = acc + (gw[i] * inv_gsum) * z[i * GROUP:(i + 1) * GROUP]
        out_ref[...] = acc

    return pl.pallas_call(
        body,
        out_shape=jax.ShapeDtypeStruct((GROUP, H), jnp.float32),
    )(agg0, agg1, dd0, dd1, b0, a0, b1, a1, fcWT, fcb, v, clWT, clb, clv)


def kernel(target_feat, edge_index_mp0, edge_index_mp1, W0, b0, a0, W1, b1, a1,
           att_fc_W, att_fc_b, att_v, attcl_fc_W, attcl_fc_b, attcl_v):
    edges = jnp.stack([edge_index_mp0.astype(jnp.int32),
                       edge_index_mp1.astype(jnp.int32)])
    edges_r = edges.reshape(2, 2, N_CHUNKS, KB, CW)
    edges_d = edges.reshape(2, 2, NS, SLAB, CW)

    zeros1d = jnp.zeros((NP // NS,), jnp.float32)
    degs = _sc_degrees(edges_d, zeros1d)
    ds0 = degs[0, 0, :N].reshape(N, 1)
    dd0 = degs[0, 1, :N].reshape(N, 1)
    ds1 = degs[1, 0, :N].reshape(N, 1)
    dd1 = degs[1, 1, :N].reshape(N, 1)

    hs0, hs1 = _tc_hs(target_feat, W0, W1, ds0, ds1)

    zrows = jnp.zeros((ROWS_T, H), jnp.float32)
    agg0, agg1 = _sc_agg(edges_r, hs0, hs1, zrows)

    out = _tc_tail(
        agg0, agg1, dd0, dd1,
        b0.reshape(1, H), a0.reshape(1, 1), b1.reshape(1, H), a1.reshape(1, 1),
        att_fc_W.T, att_fc_b.reshape(1, H), att_v.reshape(1, H),
        attcl_fc_W.T, attcl_fc_b.reshape(1, H), attcl_v.reshape(1, H))
    return out


# speedup vs baseline: 28.2589x; 1.0502x over previous
import functools

import jax
import jax.numpy as jnp
from jax import lax
from jax.experimental import pallas as pl
from jax.experimental.pallas import tpu as pltpu, tpu_sc as plsc

N = 10000
E = 320000
H = 128
GROUP = 2000
NP = 10240
NS = 16
PER_TILE = E // NS
KB = 2
CW = 80
CHUNK = KB * CW
N_ITERS = PER_TILE // CHUNK
N_CHUNKS = E // CHUNK
SLAB = PER_TILE // CW
ROWS_T = NP // NS

_mesh = plsc.VectorSubcoreMesh(core_axis_name="c", subcore_axis_name="s",
                               num_cores=2, num_subcores=NS)


def _sc_degrees(e0_d, e1_d, zeros1d):
    @functools.partial(
        pl.kernel,
        out_type=jax.ShapeDtypeStruct((2, 2, NP), jnp.float32),
        mesh=_mesh,
        scratch_types=[
            pltpu.VMEM((SLAB, CW), jnp.int32),
            pltpu.VMEM((SLAB, CW), jnp.int32),
            pltpu.VMEM((CW,), jnp.float32),
            pltpu.VMEM_SHARED((NP,), jnp.float32),
            pltpu.VMEM_SHARED((NP,), jnp.float32),
            pltpu.SemaphoreType.DMA,
            pltpu.SemaphoreType.DMA,
        ],
    )
    def deg_kernel(e0, e1, zeros_h, out, sslab, dslab, ones, deg_s, deg_d,
                   sem_i, sem_s):
        c = lax.axis_index("c")
        s = lax.axis_index("s")
        for j in range(CW // 16):
            ones[pl.ds(j * 16, 16)] = jnp.ones((16,), jnp.float32)
        seg = NP // NS
        pltpu.sync_copy(zeros_h, deg_s.at[pl.ds(seg * s, seg)])
        pltpu.sync_copy(zeros_h, deg_d.at[pl.ds(seg * s, seg)])
        plsc.subcore_barrier()

        def graph_run(e_ref):
            pltpu.async_copy(e_ref.at[0, s], sslab, sem_i)
            pltpu.async_copy(e_ref.at[1, s], dslab, sem_i)
            pltpu.make_async_copy(e_ref.at[0, s], sslab, sem_i).wait()
            pltpu.make_async_copy(e_ref.at[0, s], dslab, sem_i).wait()

            def fire(k):
                pltpu.async_copy(ones, deg_s.at[sslab.at[k]], sem_s, add=True)
                pltpu.async_copy(ones, deg_d.at[dslab.at[k]], sem_s, add=True)

            def drain(k):
                pltpu.make_async_copy(ones, deg_s.at[sslab.at[k]], sem_s).wait()
                pltpu.make_async_copy(ones, deg_d.at[dslab.at[k]], sem_s).wait()

            depth = 4
            for k in range(depth):
                fire(k)

            def body(k, carry):
                fire(k)
                drain(k - depth)
                return carry

            lax.fori_loop(depth, SLAB, body, 0)
            for k in range(depth):
                drain(SLAB - depth + k)

        for g, e_ref in ((0, e0), (1, e1)):
            @pl.when(c == g)
            def _():
                graph_run(e_ref)

        plsc.subcore_barrier()
        for g in (0, 1):
            @pl.when(c == g)
            def _():
                pltpu.sync_copy(deg_s.at[pl.ds(seg * s, seg)],
                                out.at[g, 0, pl.ds(seg * s, seg)])
                pltpu.sync_copy(deg_d.at[pl.ds(seg * s, seg)],
                                out.at[g, 1, pl.ds(seg * s, seg)])

    return deg_kernel(e0_d, e1_d, zeros1d)


def _sc_agg(e0_r, e1_r, hs0, hs1, zrows):
    @functools.partial(
        pl.kernel,
        out_type=[jax.ShapeDtypeStruct((NP, H), jnp.float32)] * 2,
        mesh=_mesh,
        scratch_types=[
            [pltpu.VMEM((KB, CW), jnp.int32)] * 3,
            [pltpu.VMEM((KB, CW), jnp.int32)] * 3,
            [pltpu.VMEM((KB, CW, H), jnp.float32)] * 2,
            pltpu.VMEM_SHARED((NP, H), jnp.float32),
            pltpu.SemaphoreType.DMA,
            pltpu.SemaphoreType.DMA,
            pltpu.SemaphoreType.DMA,
        ],
    )
    def agg_kernel(e0, e1, h0, h1, zr, out0, out1,
                   sidx, didx, rows, accum, sem_idx, sem_gat, sem_sct):
        c = lax.axis_index("c")
        s = lax.axis_index("s")
        pltpu.sync_copy(zr, accum.at[pl.ds(ROWS_T * s, ROWS_T)])
        plsc.subcore_barrier()

        def run_graph(e_ref, h_ref):
            def fire_idx(i, t):
                chunk = s * N_ITERS + i
                pltpu.async_copy(e_ref.at[0, chunk], sidx[t], sem_idx)
                pltpu.async_copy(e_ref.at[1, chunk], didx[t], sem_idx)

            def wait_idx(t):
                pltpu.make_async_copy(e_ref.at[0, 0], sidx[t], sem_idx).wait()
                pltpu.make_async_copy(e_ref.at[0, 0], didx[t], sem_idx).wait()

            def fire_gat(t, p):
                for j in range(KB):
                    pltpu.async_copy(h_ref.at[sidx[t].at[j]], rows[p].at[j],
                                     sem_gat)

            def wait_gat(p):
                for j in range(KB):
                    pltpu.make_async_copy(h_ref.at[sidx[0].at[j]],
                                          rows[p].at[j], sem_gat).wait()

            def fire_sct(t, p):
                for j in range(KB):
                    pltpu.async_copy(rows[p].at[j], accum.at[didx[t].at[j]],
                                     sem_sct, add=True)

            def wait_sct(t, p):
                for j in range(KB):
                    pltpu.make_async_copy(rows[p].at[j],
                                          accum.at[didx[t].at[j]],
                                          sem_sct).wait()

            fire_idx(0, 0)
            fire_idx(1, 1)
            wait_idx(0)
            fire_gat(0, 0)
            fire_idx(2, 2)
            wait_idx(1)
            wait_gat(0)
            fire_gat(1, 1)
            fire_sct(0, 0)

            def iter_body(i, t, p):
                wait_sct((t + 1) % 3, p)
                fire_idx(jnp.minimum(i + 1, N_ITERS - 1), (t + 1) % 3)
                wait_idx(t)
                wait_gat(1 - p)
                fire_gat(t, p)
                fire_sct((t + 2) % 3, 1 - p)

            def block(k, carry):
                i0 = 2 + 6 * k
                for off in range(6):
                    iter_body(i0 + off, (2 + off) % 3, off % 2)
                return carry

            lax.fori_loop(0, (N_ITERS - 5) // 6, block, 0)
            for i in (122, 123, 124):
                iter_body(i, i % 3, i % 2)

            wait_sct(0, 1)
            wait_idx(2)
            wait_gat(0)
            fire_sct(1, 0)
            wait_sct(1, 0)

        for g, e_ref, h_ref in ((0, e0, h0), (1, e1, h1)):
            @pl.when(c == g)
            def _():
                run_graph(e_ref, h_ref)

        plsc.subcore_barrier()
        for g, o_ref in ((0, out0), (1, out1)):
            @pl.when(c == g)
            def _():
                pltpu.sync_copy(accum.at[pl.ds(ROWS_T * s, ROWS_T)],
                                o_ref.at[pl.ds(ROWS_T * s, ROWS_T)])

    return agg_kernel(e0_r, e1_r, hs0, hs1, zrows)


def _tc_hs(x, W0, W1, ds0, ds1):
    def body(x_ref, w0_ref, w1_ref, d0_ref, d1_ref, o0_ref, o1_ref):
        xv = x_ref[...]
        for d_ref, w_ref, o_ref in ((d0_ref, w0_ref, o0_ref),
                                    (d1_ref, w1_ref, o1_ref)):
            d = d_ref[...]
            ns = jnp.where(d > 0, lax.rsqrt(jnp.maximum(d, 1.0)), 0.0)
            o_ref[...] = jnp.dot(xv * ns, w_ref[...],
                                 preferred_element_type=jnp.float32)

    return pl.pallas_call(
        body,
        out_shape=[jax.ShapeDtypeStruct((N, H), jnp.float32)] * 2,
    )(x, W0, W1, ds0, ds1)


def _tc_tail(agg0, agg1, dd0, dd1, b0, a0, b1, a1,
             fcWT, fcb, v, clWT, clb, clv):
    def body(a0_ref, a1_ref, d0_ref, d1_ref, b0_ref, s0_ref, b1_ref, s1_ref,
             fw_ref, fb_ref, v_ref, cw_ref, cb_ref, cv_ref, out_ref):
        def conv_out(ag_ref, d_ref, b_ref, slope_ref):
            d = d_ref[...]
            nd = jnp.where(d > 0, lax.rsqrt(jnp.maximum(d, 1.0)), 0.0)
            y = ag_ref[0:N] * nd + b_ref[...]
            return jnp.where(y >= 0, y, slope_ref[...] * y)

        e0 = conv_out(a0_ref, d0_ref, b0_ref, s0_ref)
        e1 = conv_out(a1_ref, d1_ref, b1_ref, s1_ref)
        fw = fw_ref[...]
        fb = fb_ref[...]
        vv = v_ref[...]
        inv_n = jnp.float32(1.0 / N)
        l0 = jnp.sum(jnp.tanh(jnp.dot(e0, fw, preferred_element_type=jnp.float32)
                              + fb) * vv) * inv_n
        l1 = jnp.sum(jnp.tanh(jnp.dot(e1, fw, preferred_element_type=jnp.float32)
                              + fb) * vv) * inv_n
        m = jnp.maximum(l0, l1)
        w0 = jnp.exp(l0 - m)
        w1 = jnp.exp(l1 - m)
        inv_sum = 1.0 / (w0 + w1)
        z = (w0 * inv_sum) * e0 + (w1 * inv_sum) * e1

        tz = jnp.tanh(jnp.dot(z, cw_ref[...], preferred_element_type=jnp.float32)
                      + cb_ref[...])
        cvv = cv_ref[...]
        inv_g = jnp.float32(1.0 / GROUP)
        gl = [jnp.sum(tz[i * GROUP:(i + 1) * GROUP] * cvv) * inv_g
              for i in range(N // GROUP)]
        gm = gl[0]
        for t in gl[1:]:
            gm = jnp.maximum(gm, t)
        gw = [jnp.exp(t - gm) for t in gl]
        gsum = gw[0]
        for t in gw[1:]:
            gsum = gsum + t
        inv_gsum = 1.0 / gsum
        acc = (gw[0] * inv_gsum) * z[0:GROUP]
        for i in range(1, N // GROUP):
            acc = acc + (gw[i] * inv_gsum) * z[i * GROUP:(i + 1) * GROUP]
        out_ref[...] = acc

    return pl.pallas_call(
        body,
        out_shape=jax.ShapeDtypeStruct((GROUP, H), jnp.float32),
    )(agg0, agg1, dd0, dd1, b0, a0, b1, a1, fcWT, fcb, v, clWT, clb, clv)


def kernel(target_feat, edge_index_mp0, edge_index_mp1, W0, b0, a0, W1, b1, a1,
           att_fc_W, att_fc_b, att_v, attcl_fc_W, attcl_fc_b, attcl_v):
    ei0 = edge_index_mp0.astype(jnp.int32)
    ei1 = edge_index_mp1.astype(jnp.int32)
    e0_r = ei0.reshape(2, N_CHUNKS, KB, CW)
    e1_r = ei1.reshape(2, N_CHUNKS, KB, CW)
    e0_d = ei0.reshape(2, NS, SLAB, CW)
    e1_d = ei1.reshape(2, NS, SLAB, CW)

    zeros1d = jnp.zeros((NP // NS,), jnp.float32)
    degs = _sc_degrees(e0_d, e1_d, zeros1d)
    ds0 = degs[0, 0, :N].reshape(N, 1)
    dd0 = degs[0, 1, :N].reshape(N, 1)
    ds1 = degs[1, 0, :N].reshape(N, 1)
    dd1 = degs[1, 1, :N].reshape(N, 1)

    hs0, hs1 = _tc_hs(target_feat, W0, W1, ds0, ds1)

    zrows = jnp.zeros((ROWS_T, H), jnp.float32)
    agg0, agg1 = _sc_agg(e0_r, e1_r, hs0, hs1, zrows)

    out = _tc_tail(
        agg0, agg1, dd0, dd1,
        b0.reshape(1, H), a0.reshape(1, 1), b1.reshape(1, H), a1.reshape(1, 1),
        att_fc_W.T, att_fc_b.reshape(1, H), att_v.reshape(1, H),
        attcl_fc_W.T, attcl_fc_b.reshape(1, H), attcl_v.reshape(1, H))
    return out
